# baseline (speedup 1.0000x reference)
import jax
import jax.numpy as jnp
from jax.experimental import pallas as pl

_B, _P, _F, _K, _L = 128, 64, 64, 25, 3
_E = 16
_E2 = _E // 2


def _gnn_kernel(x_ref, w1_ref, b1_ref, w2_ref, b2_ref, zbd_ref, out_ref):
    x = x_ref[...]
    x2 = jnp.sum(x * x, axis=-1)
    xb = x.astype(jnp.bfloat16)
    xxt = jnp.stack(
        [jax.lax.dot_general(xb[e], xb[e], (((1,), (1,)), ((), ())),
                             preferred_element_type=jnp.float32)
         for e in range(_E)], axis=0)
    d = x2[:, :, None] + x2[:, None, :] - 2.0 * xxt
    ii = jax.lax.broadcasted_iota(jnp.int32, (_P, _P), 0)
    jj = jax.lax.broadcasted_iota(jnp.int32, (_P, _P), 1)
    d = d + jnp.where(ii == jj, jnp.float32(1e9), jnp.float32(0.0))[None]

    d2 = jnp.concatenate([d[:_E2], d[_E2:]], axis=-1)
    jrow = jax.lax.broadcasted_iota(jnp.int32, (_P, 2 * _F), 0)
    lcol = jax.lax.broadcasted_iota(jnp.int32, (_P, 2 * _F), 1)
    tie2 = jnp.where((lcol & (_F - 1)) < jrow, jnp.float32(1.0),
                     jnp.float32(0.0))
    mask_slices = []
    for e in range(_E2):
        dj2 = jnp.concatenate(
            [jnp.broadcast_to(d[e][:, :, None], (_P, _P, _F)),
             jnp.broadcast_to(d[_E2 + e][:, :, None], (_P, _P, _F))],
            axis=-1)
        dk2 = d2[e][:, None, :]
        beat = jnp.clip((dj2 - dk2) * jnp.float32(1e38) + tie2[None],
                        jnp.float32(0.0), jnp.float32(1.0))
        rank2 = jnp.dot(beat.reshape(_P * _P, 2 * _F), zbd_ref[0])
        mask_slices.append(jnp.where(rank2 < _K, jnp.float32(1.0),
                                     jnp.float32(0.0)))
    mask2 = jnp.stack(mask_slices).reshape(_E2, _P, _P, 2 * _F)

    xl2 = jnp.concatenate([x[:_E2].reshape(_E2 * _P, _F),
                           x[_E2:].reshape(_E2 * _P, _F)], axis=-1)
    for l in range(_L):
        ab2 = jnp.dot(xl2, w1_ref[l], precision=jax.lax.Precision.HIGHEST)
        a2 = (ab2[:, :2 * _F] + b1_ref[l]).reshape(_E2, _P, 2 * _F)
        bm2 = ab2[:, 2 * _F:].reshape(_E2, _P, 2 * _F)
        s2 = jnp.stack(
            [jnp.sum(mask2[e] * jax.nn.relu(a2[e][:, None, :]
                                            + bm2[e][None, :, :]), axis=1)
             for e in range(_E2)], axis=0)
        xl2 = jnp.dot(s2.reshape(_E2 * _P, 2 * _F), w2_ref[l],
                      precision=jax.lax.Precision.HIGHEST) + _K * b2_ref[l]
    out = jnp.concatenate([xl2[:, :_F].reshape(_E2, _P, _F),
                           xl2[:, _F:].reshape(_E2, _P, _F)], axis=0)
    out_ref[...] = out


def _blockdiag(m):
    z = jnp.zeros_like(m)
    return jnp.concatenate(
        [jnp.concatenate([m, z], axis=1), jnp.concatenate([z, m], axis=1)],
        axis=0)


def kernel(random_vector, W1_0, b1_0, W2_0, b2_0, W1_1, b1_1, W2_1, b2_1,
           W1_2, b1_2, W2_2, b2_2):
    w1bd, b1t, w2bd, b2t = [], [], [], []
    for w1, b1, w2, b2 in ((W1_0, b1_0, W2_0, b2_0), (W1_1, b1_1, W2_1, b2_1),
                           (W1_2, b1_2, W2_2, b2_2)):
        w1d = w1[:_F] - w1[_F:]
        w1b = w1[_F:]
        w1bd.append(jnp.concatenate([_blockdiag(w1d), _blockdiag(w1b)],
                                    axis=1))
        b1t.append(jnp.tile(b1, 2).reshape(1, 2 * _F))
        w2bd.append(_blockdiag(w2))
        b2t.append(jnp.tile(b2, 2).reshape(1, 2 * _F))
    w1bd = jnp.stack(w1bd)
    b1t = jnp.stack(b1t)
    w2bd = jnp.stack(w2bd)
    b2t = jnp.stack(b2t)
    ones = jnp.ones((_F, _F), jnp.float32)
    zbd = _blockdiag(ones)[None]
    return pl.pallas_call(
        _gnn_kernel,
        grid=(_B // _E,),
        in_specs=[
            pl.BlockSpec((_E, _P, _F), lambda i: (i, 0, 0)),
            pl.BlockSpec((_L, 2 * _F, 4 * _F), lambda i: (0, 0, 0)),
            pl.BlockSpec((_L, 1, 2 * _F), lambda i: (0, 0, 0)),
            pl.BlockSpec((_L, 2 * _F, 2 * _F), lambda i: (0, 0, 0)),
            pl.BlockSpec((_L, 1, 2 * _F), lambda i: (0, 0, 0)),
            pl.BlockSpec((1, 2 * _F, 2 * _F), lambda i: (0, 0, 0)),
        ],
        out_specs=pl.BlockSpec((_E, _P, _F), lambda i: (i, 0, 0)),
        out_shape=jax.ShapeDtypeStruct((_B, _P, _F), jnp.float32),
    )(random_vector, w1bd, b1t, w2bd, b2t, zbd)


# speedup vs baseline: 11.5304x; 11.5304x over previous
import jax
import jax.numpy as jnp
from jax.experimental import pallas as pl

_B, _P, _F, _K, _L = 128, 64, 64, 25, 3
_E = 16
_E2 = _E // 2


def _gnn_kernel(x_ref, w1_ref, b1_ref, w2_ref, b2_ref, zbd_ref, out_ref):
    x = x_ref[...]
    x2 = jnp.sum(x * x, axis=-1)
    xb = x.astype(jnp.bfloat16)
    xxt = jnp.stack(
        [jax.lax.dot_general(xb[e], xb[e], (((1,), (1,)), ((), ())),
                             preferred_element_type=jnp.float32)
         for e in range(_E)], axis=0)
    d = x2[:, :, None] + x2[:, None, :] - 2.0 * xxt
    ii = jax.lax.broadcasted_iota(jnp.int32, (_P, _P), 0)
    jj = jax.lax.broadcasted_iota(jnp.int32, (_P, _P), 1)
    d = d + jnp.where(ii == jj, jnp.float32(1e9), jnp.float32(0.0))[None]

    d2 = jnp.concatenate([d[:_E2], d[_E2:]], axis=-1)
    jrow = jax.lax.broadcasted_iota(jnp.int32, (_P, 2 * _F), 0)
    lcol = jax.lax.broadcasted_iota(jnp.int32, (_P, 2 * _F), 1)
    tie2 = jnp.where((lcol & (_F - 1)) < jrow, jnp.float32(1.0),
                     jnp.float32(0.0))
    mask_slices = []
    for e in range(_E2):
        dj2 = jnp.concatenate(
            [jnp.broadcast_to(d[e][:, :, None], (_P, _P, _F)),
             jnp.broadcast_to(d[_E2 + e][:, :, None], (_P, _P, _F))],
            axis=-1)
        dk2 = d2[e][:, None, :]
        beat = jnp.clip((dj2 - dk2) * jnp.float32(1e38) + tie2[None],
                        jnp.float32(0.0), jnp.float32(1.0))
        rank2 = jnp.dot(beat.reshape(_P * _P, 2 * _F), zbd_ref[0])
        mask_slices.append(jnp.where(rank2 < _K, jnp.float32(1.0),
                                     jnp.float32(0.0)))
    mask2 = jnp.stack(mask_slices).reshape(_E2, _P, _P, 2 * _F)

    xl2 = jnp.concatenate([x[:_E2].reshape(_E2 * _P, _F),
                           x[_E2:].reshape(_E2 * _P, _F)], axis=-1)
    for l in range(_L):
        ab2 = jnp.dot(xl2, w1_ref[l], precision=jax.lax.Precision.HIGHEST)
        a2 = (ab2[:, :2 * _F] + b1_ref[l]).reshape(_E2, _P, 2 * _F)
        bm2 = ab2[:, 2 * _F:].reshape(_E2, _P, 2 * _F)
        pair = a2[:, :, None, :] + bm2[:, None, :, :]
        s2 = jnp.sum(mask2 * jax.nn.relu(pair), axis=2)
        xl2 = jnp.dot(s2.reshape(_E2 * _P, 2 * _F), w2_ref[l],
                      precision=jax.lax.Precision.HIGHEST) + _K * b2_ref[l]
    out = jnp.concatenate([xl2[:, :_F].reshape(_E2, _P, _F),
                           xl2[:, _F:].reshape(_E2, _P, _F)], axis=0)
    out_ref[...] = out


def _blockdiag(m):
    z = jnp.zeros_like(m)
    return jnp.concatenate(
        [jnp.concatenate([m, z], axis=1), jnp.concatenate([z, m], axis=1)],
        axis=0)


def kernel(random_vector, W1_0, b1_0, W2_0, b2_0, W1_1, b1_1, W2_1, b2_1,
           W1_2, b1_2, W2_2, b2_2):
    w1bd, b1t, w2bd, b2t = [], [], [], []
    for w1, b1, w2, b2 in ((W1_0, b1_0, W2_0, b2_0), (W1_1, b1_1, W2_1, b2_1),
                           (W1_2, b1_2, W2_2, b2_2)):
        w1d = w1[:_F] - w1[_F:]
        w1b = w1[_F:]
        w1bd.append(jnp.concatenate([_blockdiag(w1d), _blockdiag(w1b)],
                                    axis=1))
        b1t.append(jnp.tile(b1, 2).reshape(1, 2 * _F))
        w2bd.append(_blockdiag(w2))
        b2t.append(jnp.tile(b2, 2).reshape(1, 2 * _F))
    w1bd = jnp.stack(w1bd)
    b1t = jnp.stack(b1t)
    w2bd = jnp.stack(w2bd)
    b2t = jnp.stack(b2t)
    ones = jnp.ones((_F, _F), jnp.float32)
    zbd = _blockdiag(ones)[None]
    return pl.pallas_call(
        _gnn_kernel,
        grid=(_B // _E,),
        in_specs=[
            pl.BlockSpec((_E, _P, _F), lambda i: (i, 0, 0)),
            pl.BlockSpec((_L, 2 * _F, 4 * _F), lambda i: (0, 0, 0)),
            pl.BlockSpec((_L, 1, 2 * _F), lambda i: (0, 0, 0)),
            pl.BlockSpec((_L, 2 * _F, 2 * _F), lambda i: (0, 0, 0)),
            pl.BlockSpec((_L, 1, 2 * _F), lambda i: (0, 0, 0)),
            pl.BlockSpec((1, 2 * _F, 2 * _F), lambda i: (0, 0, 0)),
        ],
        out_specs=pl.BlockSpec((_E, _P, _F), lambda i: (i, 0, 0)),
        out_shape=jax.ShapeDtypeStruct((_B, _P, _F), jnp.float32),
    )(random_vector, w1bd, b1t, w2bd, b2t, zbd)
